# Initial kernel scaffold; baseline (speedup 1.0000x reference)
import jax
import jax.numpy as jnp
from jax.experimental import pallas as pl


def kernel(x, idx_cluster, token_weight, cluster_num):
    raise NotImplementedError("write your pallas kernel here")


# speedup vs baseline: 7.3419x; 7.3419x over previous
import jax
import jax.numpy as jnp
from jax import lax
from jax.experimental import pallas as pl
from jax.experimental.pallas import tpu as pltpu
from jax.experimental.pallas import tpu_sc as plsc

L = 16

B, N, C = 16, 4096, 64
CLUSTER = 1024
NC, NS = 2, 16
B_PER_SC = B // NC
SEGS_SC = B_PER_SC * CLUSTER
TOK_SC = B_PER_SC * N
TOK_TILE = TOK_SC // NS
CHUNK = 128
N_CHUNK = TOK_TILE // CHUNK
W = CHUNK
SEGS_TILE = SEGS_SC // NS


def _sc_kernel(x_hbm, idx_hbm, tw_hbm, out_hbm, acc_sh, xbuf, xwbuf, idxcur, twv):
    c = lax.axis_index("c")
    s = lax.axis_index("s")
    b_local = s // 2
    zero16 = jnp.zeros((L,), jnp.float32)
    lane = lax.iota(jnp.int32, L)
    onehot0 = jnp.where(lane == 0, 1.0, 0.0).astype(jnp.float32)

    def zrow(i, _):
        r = i // (W // L)
        f = i % (W // L)
        xwbuf[r, pl.ds(f * L, L)] = zero16
        return 0
    lax.fori_loop(0, CHUNK * (W // L), zrow, 0)
    for k in range(SEGS_TILE // CHUNK):
        pltpu.sync_copy(xwbuf, acc_sh.at[pl.ds(s * SEGS_TILE + k * CHUNK, CHUNK)])

    row0 = c * (TOK_SC // CHUNK) + s * N_CHUNK
    pltpu.sync_copy(tw_hbm.at[pl.ds(row0, N_CHUNK)], twv)
    boff = b_local * CLUSTER

    plsc.subcore_barrier()

    tok0 = c * TOK_SC + s * TOK_TILE
    for j in range(N_CHUNK):
        pltpu.sync_copy(x_hbm.at[pl.ds(tok0 + j * CHUNK, CHUNK)], xbuf)
        pltpu.sync_copy(idx_hbm.at[row0 + j], idxcur)

        def oadd(g, _):
            idxcur[pl.ds(g * L, L)] = idxcur[pl.ds(g * L, L)] + boff
            return 0
        lax.fori_loop(0, CHUNK // L, oadd, 0)

        def body(g, _):
            tv = twv[j, pl.ds(g * L, L)]
            for tk in range(L):
                t = g * L + tk
                wv = jnp.full((L,), tv[tk], jnp.float32)
                for f in range(C // L):
                    xwbuf[t, pl.ds(f * L, L)] = xbuf[t, pl.ds(f * L, L)] * wv
                xwbuf[t, pl.ds(C, L)] = wv * onehot0
            return 0
        lax.fori_loop(0, CHUNK // L, body, 0)

        pltpu.sync_copy(xwbuf, acc_sh.at[idxcur], add=True)

    plsc.subcore_barrier()

    for h in range(SEGS_TILE // CHUNK):
        seg0 = s * SEGS_TILE + h * CHUNK
        pltpu.sync_copy(acc_sh.at[pl.ds(seg0, CHUNK)], xwbuf)

        def nbody(t, _):
            wrow = xwbuf[t, pl.ds(C, L)]
            wv = jnp.full((L,), wrow[0], jnp.float32)
            rv = 1.0 / (wv + 1e-6)
            for f in range(C // L):
                xbuf[t, pl.ds(f * L, L)] = xwbuf[t, pl.ds(f * L, L)] * rv
            return 0
        lax.fori_loop(0, CHUNK, nbody, 0)

        pltpu.sync_copy(xbuf, out_hbm.at[pl.ds(c * SEGS_SC + seg0, CHUNK)])


@jax.jit
def _ctm_merge(x2, idx2, tw2):
    mesh = plsc.VectorSubcoreMesh(core_axis_name="c", subcore_axis_name="s")
    run = pl.kernel(
        _sc_kernel,
        out_type=jax.ShapeDtypeStruct((B * CLUSTER, C), jnp.float32),
        mesh=mesh,
        scratch_types=[
            pltpu.VMEM_SHARED((SEGS_SC, W), jnp.float32),
            pltpu.VMEM((CHUNK, C), jnp.float32),
            pltpu.VMEM((CHUNK, W), jnp.float32),
            pltpu.VMEM((CHUNK,), jnp.int32),
            pltpu.VMEM((N_CHUNK, CHUNK), jnp.float32),
        ],
    )
    return run(x2, idx2, tw2)


def kernel(x, idx_cluster, token_weight, cluster_num):
    b, n, c = x.shape
    x2 = x.reshape(b * n, c)
    idx2 = idx_cluster.reshape(b * n // CHUNK, CHUNK)
    tw2 = token_weight.reshape(b * n // CHUNK, CHUNK)
    out = _ctm_merge(x2, idx2, tw2)
    return out.reshape(b, CLUSTER, c)


# speedup vs baseline: 8.4169x; 1.1464x over previous
import jax
import jax.numpy as jnp
from jax import lax
from jax.experimental import pallas as pl
from jax.experimental.pallas import tpu as pltpu
from jax.experimental.pallas import tpu_sc as plsc

L = 16

B, N, C = 16, 4096, 64
CLUSTER = 1024
NC, NS = 2, 16
B_PER_SC = B // NC
SEGS_SC = B_PER_SC * CLUSTER
TOK_SC = B_PER_SC * N
TOK_TILE = TOK_SC // NS
SUB = 64
NSUB = TOK_TILE // SUB
W = 128
SEGS_TILE = SEGS_SC // NS


def _sc_kernel(x_hbm, idx_hbm, tw_hbm, out_hbm, acc_sh,
               xb0, xb1, xw0, xw1, idxb0, idxb1, twb0, twb1, idxs0, idxs1,
               xsem0, xsem1, ssem0, ssem1):
    c = lax.axis_index("c")
    s = lax.axis_index("s")
    b_local = s // (NS // B_PER_SC)
    boff = b_local * CLUSTER
    tok0 = c * TOK_SC + s * TOK_TILE
    zero16 = jnp.zeros((L,), jnp.float32)
    lane = lax.iota(jnp.int32, L)
    onehot0 = jnp.where(lane == 0, 1.0, 0.0).astype(jnp.float32)

    xb = (xb0, xb1)
    xw = (xw0, xw1)
    idxb = (idxb0, idxb1)
    twb = (twb0, twb1)
    idxs = (idxs0, idxs1)
    xsem = (xsem0, xsem1)
    ssem = (ssem0, ssem1)

    def start_load(j, p):
        base = tok0 + j * SUB
        pltpu.async_copy(x_hbm.at[pl.ds(base, SUB)], xb[p], xsem[p])
        pltpu.async_copy(idx_hbm.at[pl.ds(base, SUB)], idxb[p], xsem[p])
        pltpu.async_copy(tw_hbm.at[pl.ds(base, SUB)], twb[p], xsem[p])

    def wait_load(j, p):
        base = tok0 + j * SUB
        pltpu.make_async_copy(x_hbm.at[pl.ds(base, SUB)], xb[p], xsem[p]).wait()
        pltpu.make_async_copy(idx_hbm.at[pl.ds(base, SUB)], idxb[p], xsem[p]).wait()
        pltpu.make_async_copy(tw_hbm.at[pl.ds(base, SUB)], twb[p], xsem[p]).wait()

    def wait_scatter(p):
        pltpu.make_async_copy(xw[p], acc_sh.at[idxs[p]], ssem[p]).wait()

    start_load(0, 0)
    start_load(1, 1)

    def zrow(i, _):
        r = i // (W // L)
        f = i % (W // L)
        xw0[r, pl.ds(f * L, L)] = zero16
        xw1[r, pl.ds(f * L, L)] = zero16
        return 0
    lax.fori_loop(0, SUB * (W // L), zrow, 0)
    for k in range(SEGS_TILE // SUB):
        pltpu.sync_copy(xw0, acc_sh.at[pl.ds(s * SEGS_TILE + k * SUB, SUB)])

    plsc.subcore_barrier()

    def step(g, _):
        for p in range(2):
            j = 2 * g + p

            @pl.when(g > 0)
            def _():
                wait_scatter(p)
            wait_load(j, p)

            def body(g16, _):
                tv = twb[p][pl.ds(g16 * L, L)]
                idxs[p][pl.ds(g16 * L, L)] = idxb[p][pl.ds(g16 * L, L)] + boff
                for tk in range(L):
                    t = g16 * L + tk
                    wv = jnp.full((L,), tv[tk], jnp.float32)
                    for f in range(C // L):
                        xw[p][t, pl.ds(f * L, L)] = xb[p][t, pl.ds(f * L, L)] * wv
                    xw[p][t, pl.ds(C, L)] = wv * onehot0
                return 0
            lax.fori_loop(0, SUB // L, body, 0)

            @pl.when(g < (NSUB // 2) - 1)
            def _():
                start_load(j + 2, p)
            pltpu.async_copy(xw[p], acc_sh.at[idxs[p]], ssem[p], add=True)
        return 0
    lax.fori_loop(0, NSUB // 2, step, 0)
    wait_scatter(0)
    wait_scatter(1)

    plsc.subcore_barrier()

    for h in range(SEGS_TILE // SUB):
        seg0 = s * SEGS_TILE + h * SUB
        pltpu.sync_copy(acc_sh.at[pl.ds(seg0, SUB)], xw0)

        def nbody(t, _):
            wrow = xw0[t, pl.ds(C, L)]
            wv = jnp.full((L,), wrow[0], jnp.float32)
            rv = 1.0 / (wv + 1e-6)
            for f in range(C // L):
                xb0[t, pl.ds(f * L, L)] = xw0[t, pl.ds(f * L, L)] * rv
            return 0
        lax.fori_loop(0, SUB, nbody, 0)

        pltpu.sync_copy(xb0, out_hbm.at[pl.ds(c * SEGS_SC + seg0, SUB)])


@jax.jit
def _ctm_merge(x2, idx1, tw1):
    mesh = plsc.VectorSubcoreMesh(core_axis_name="c", subcore_axis_name="s")
    run = pl.kernel(
        _sc_kernel,
        out_type=jax.ShapeDtypeStruct((B * CLUSTER, C), jnp.float32),
        mesh=mesh,
        scratch_types=[
            pltpu.VMEM_SHARED((SEGS_SC, W), jnp.float32),
            pltpu.VMEM((SUB, C), jnp.float32),
            pltpu.VMEM((SUB, C), jnp.float32),
            pltpu.VMEM((SUB, W), jnp.float32),
            pltpu.VMEM((SUB, W), jnp.float32),
            pltpu.VMEM((SUB,), jnp.int32),
            pltpu.VMEM((SUB,), jnp.int32),
            pltpu.VMEM((SUB,), jnp.float32),
            pltpu.VMEM((SUB,), jnp.float32),
            pltpu.VMEM((SUB,), jnp.int32),
            pltpu.VMEM((SUB,), jnp.int32),
            pltpu.SemaphoreType.DMA,
            pltpu.SemaphoreType.DMA,
            pltpu.SemaphoreType.DMA,
            pltpu.SemaphoreType.DMA,
        ],
    )
    return run(x2, idx1, tw1)


def kernel(x, idx_cluster, token_weight, cluster_num):
    b, n, c = x.shape
    x2 = x.reshape(b * n, c)
    idx1 = idx_cluster.reshape(b * n)
    tw1 = token_weight.reshape(b * n)
    out = _ctm_merge(x2, idx1, tw1)
    return out.reshape(b, CLUSTER, c)


# speedup vs baseline: 10.5391x; 1.2521x over previous
import jax
import jax.numpy as jnp
from jax import lax
from jax.experimental import pallas as pl
from jax.experimental.pallas import tpu as pltpu
from jax.experimental.pallas import tpu_sc as plsc

L = 16

B, N, C = 16, 4096, 64
CLUSTER = 1024
NC, NS = 2, 16
B_PER_SC = B // NC
SEGS_SC = B_PER_SC * CLUSTER
TOK_SC = B_PER_SC * N
TOK_TILE = TOK_SC // NS
SUB = 64
NSUB = TOK_TILE // SUB
W = 128
SEGS_TILE = SEGS_SC // NS


def _sc_kernel(x_hbm, idx_hbm, tw_hbm, out_hbm, acc_sh,
               xb0, xb1, xw0, xw1, idxb0, idxb1, twb0, twb1, idxs0, idxs1,
               xsem0, xsem1, ssem0, ssem1):
    c = lax.axis_index("c")
    s = lax.axis_index("s")
    b_local = s // (NS // B_PER_SC)
    boff = b_local * CLUSTER
    tok0 = c * TOK_SC + s * TOK_TILE
    zero16 = jnp.zeros((L,), jnp.float32)
    lane = lax.iota(jnp.int32, L)
    onehot0 = jnp.where(lane == 0, 1.0, 0.0).astype(jnp.float32)

    xb = (xb0, xb1)
    xw = (xw0, xw1)
    idxb = (idxb0, idxb1)
    twb = (twb0, twb1)
    idxs = (idxs0, idxs1)
    xsem = (xsem0, xsem1)
    ssem = (ssem0, ssem1)

    def start_load(j, p):
        base = tok0 + j * SUB
        pltpu.async_copy(x_hbm.at[pl.ds(base, SUB)], xb[p], xsem[p])
        pltpu.async_copy(idx_hbm.at[pl.ds(base, SUB)], idxb[p], xsem[p])
        pltpu.async_copy(tw_hbm.at[pl.ds(base, SUB)], twb[p], xsem[p])

    def wait_load(j, p):
        base = tok0 + j * SUB
        pltpu.make_async_copy(x_hbm.at[pl.ds(base, SUB)], xb[p], xsem[p]).wait()
        pltpu.make_async_copy(idx_hbm.at[pl.ds(base, SUB)], idxb[p], xsem[p]).wait()
        pltpu.make_async_copy(tw_hbm.at[pl.ds(base, SUB)], twb[p], xsem[p]).wait()

    def wait_scatter(p):
        pltpu.make_async_copy(xw[p], acc_sh.at[idxs[p]], ssem[p]).wait()

    start_load(0, 0)
    start_load(1, 1)

    def zrow(i, _):
        r = i // (W // L)
        f = i % (W // L)
        xw0[r, pl.ds(f * L, L)] = zero16
        xw1[r, pl.ds(f * L, L)] = zero16
        return 0
    lax.fori_loop(0, SUB * (W // L), zrow, 0)
    for k in range(SEGS_TILE // SUB):
        pltpu.sync_copy(xw0, acc_sh.at[pl.ds(s * SEGS_TILE + k * SUB, SUB)])

    plsc.subcore_barrier()

    def step(g, _):
        for p in range(2):
            j = 2 * g + p

            @pl.when(g > 0)
            def _():
                wait_scatter(p)
            wait_load(j, p)

            @plsc.parallel_loop(0, SUB // L, unroll=2)
            def body(g16):
                tv = twb[p][pl.ds(g16 * L, L)]
                idxs[p][pl.ds(g16 * L, L)] = idxb[p][pl.ds(g16 * L, L)] + boff
                for tk in range(L):
                    t = g16 * L + tk
                    wv = jnp.full((L,), tv[tk], jnp.float32)
                    for f in range(C // L):
                        xw[p][t, pl.ds(f * L, L)] = xb[p][t, pl.ds(f * L, L)] * wv
                    xw[p][t, pl.ds(C, L)] = wv * onehot0

            @pl.when(g < (NSUB // 2) - 1)
            def _():
                start_load(j + 2, p)
            pltpu.async_copy(xw[p], acc_sh.at[idxs[p]], ssem[p], add=True)
        return 0
    lax.fori_loop(0, NSUB // 2, step, 0)
    wait_scatter(0)
    wait_scatter(1)

    plsc.subcore_barrier()

    NH = SEGS_TILE // SUB

    def ep_read(h, p):
        seg0 = s * SEGS_TILE + h * SUB
        return pltpu.make_async_copy(acc_sh.at[pl.ds(seg0, SUB)], xw[p], xsem[p])

    def ep_write(h, p):
        seg0 = s * SEGS_TILE + h * SUB
        return pltpu.make_async_copy(xb[p], out_hbm.at[pl.ds(c * SEGS_SC + seg0, SUB)], ssem[p])

    ep_read(0, 0).start()
    ep_read(1, 1).start()
    for h in range(NH):
        p = h & 1
        ep_read(h, p).wait()
        if h >= 2:
            ep_write(h - 2, p).wait()

        @plsc.parallel_loop(0, SUB, unroll=2)
        def nbody(t):
            wrow = xw[p][t, pl.ds(C, L)]
            wv = jnp.full((L,), wrow[0], jnp.float32)
            rv = 1.0 / (wv + 1e-6)
            for f in range(C // L):
                xb[p][t, pl.ds(f * L, L)] = xw[p][t, pl.ds(f * L, L)] * rv

        if h < NH - 2:
            ep_read(h + 2, p).start()
        ep_write(h, p).start()
    ep_write(NH - 2, 0).wait()
    ep_write(NH - 1, 1).wait()


@jax.jit
def _ctm_merge(x2, idx1, tw1):
    mesh = plsc.VectorSubcoreMesh(core_axis_name="c", subcore_axis_name="s")
    run = pl.kernel(
        _sc_kernel,
        out_type=jax.ShapeDtypeStruct((B * CLUSTER, C), jnp.float32),
        mesh=mesh,
        scratch_types=[
            pltpu.VMEM_SHARED((SEGS_SC, W), jnp.float32),
            pltpu.VMEM((SUB, C), jnp.float32),
            pltpu.VMEM((SUB, C), jnp.float32),
            pltpu.VMEM((SUB, W), jnp.float32),
            pltpu.VMEM((SUB, W), jnp.float32),
            pltpu.VMEM((SUB,), jnp.int32),
            pltpu.VMEM((SUB,), jnp.int32),
            pltpu.VMEM((SUB,), jnp.float32),
            pltpu.VMEM((SUB,), jnp.float32),
            pltpu.VMEM((SUB,), jnp.int32),
            pltpu.VMEM((SUB,), jnp.int32),
            pltpu.SemaphoreType.DMA,
            pltpu.SemaphoreType.DMA,
            pltpu.SemaphoreType.DMA,
            pltpu.SemaphoreType.DMA,
        ],
    )
    return run(x2, idx1, tw1)


def kernel(x, idx_cluster, token_weight, cluster_num):
    b, n, c = x.shape
    x2 = x.reshape(b * n, c)
    idx1 = idx_cluster.reshape(b * n)
    tw1 = token_weight.reshape(b * n)
    out = _ctm_merge(x2, idx1, tw1)
    return out.reshape(b, CLUSTER, c)


# speedup vs baseline: 10.7337x; 1.0185x over previous
import jax
import jax.numpy as jnp
from jax import lax
from jax.experimental import pallas as pl
from jax.experimental.pallas import tpu as pltpu
from jax.experimental.pallas import tpu_sc as plsc

L = 16

B, N, C = 16, 4096, 64
CLUSTER = 1024
NC, NS = 2, 16
B_PER_SC = B // NC
SEGS_SC = B_PER_SC * CLUSTER
TOK_SC = B_PER_SC * N
TOK_TILE = TOK_SC // NS
SUB = 64
NSUB = TOK_TILE // SUB
W = 128
SEGS_TILE = SEGS_SC // NS


def _sc_kernel(x_hbm, idx_hbm, tw_hbm, out_hbm, acc_sh,
               xb0, xb1, xw0, xw1, idxb0, idxb1, twb0, twb1, idxs0, idxs1,
               xsem0, xsem1, ssem0, ssem1):
    c = lax.axis_index("c")
    s = lax.axis_index("s")
    b_local = s // (NS // B_PER_SC)
    boff = b_local * CLUSTER
    tok0 = c * TOK_SC + s * TOK_TILE
    zero16 = jnp.zeros((L,), jnp.float32)
    lane = lax.iota(jnp.int32, L)
    onehot0 = jnp.where(lane == 0, 1.0, 0.0).astype(jnp.float32)

    xb = (xb0, xb1)
    xw = (xw0, xw1)
    idxb = (idxb0, idxb1)
    twb = (twb0, twb1)
    idxs = (idxs0, idxs1)
    xsem = (xsem0, xsem1)
    ssem = (ssem0, ssem1)

    def start_load(j, p):
        base = tok0 + j * SUB
        pltpu.async_copy(x_hbm.at[pl.ds(base, SUB)], xb[p], xsem[p])
        pltpu.async_copy(idx_hbm.at[pl.ds(base, SUB)], idxb[p], xsem[p])
        pltpu.async_copy(tw_hbm.at[pl.ds(base, SUB)], twb[p], xsem[p])

    def wait_load(j, p):
        base = tok0 + j * SUB
        pltpu.make_async_copy(x_hbm.at[pl.ds(base, SUB)], xb[p], xsem[p]).wait()
        pltpu.make_async_copy(idx_hbm.at[pl.ds(base, SUB)], idxb[p], xsem[p]).wait()
        pltpu.make_async_copy(tw_hbm.at[pl.ds(base, SUB)], twb[p], xsem[p]).wait()

    def wait_scatter(p):
        pltpu.make_async_copy(xw[p], acc_sh.at[idxs[p]], ssem[p]).wait()

    start_load(0, 0)
    start_load(1, 1)

    @plsc.parallel_loop(0, SUB * (W // L), unroll=4)
    def zrow(i):
        r = i // (W // L)
        f = i % (W // L)
        xw0[r, pl.ds(f * L, L)] = zero16
        xw1[r, pl.ds(f * L, L)] = zero16

    zcopies = [
        pltpu.make_async_copy(
            xw0, acc_sh.at[pl.ds(s * SEGS_TILE + k * SUB, SUB)], ssem0)
        for k in range(SEGS_TILE // SUB)
    ]
    for zc in zcopies:
        zc.start()
    for zc in zcopies:
        zc.wait()

    plsc.subcore_barrier()

    def step(g, _):
        for p in range(2):
            j = 2 * g + p

            @pl.when(g > 0)
            def _():
                wait_scatter(p)
            wait_load(j, p)

            @plsc.parallel_loop(0, SUB // L, unroll=4)
            def body(g16):
                tv = twb[p][pl.ds(g16 * L, L)]
                idxs[p][pl.ds(g16 * L, L)] = idxb[p][pl.ds(g16 * L, L)] + boff
                for tk in range(L):
                    t = g16 * L + tk
                    wv = jnp.full((L,), tv[tk], jnp.float32)
                    for f in range(C // L):
                        xw[p][t, pl.ds(f * L, L)] = xb[p][t, pl.ds(f * L, L)] * wv
                    xw[p][t, pl.ds(C, L)] = wv * onehot0

            @pl.when(g < (NSUB // 2) - 1)
            def _():
                start_load(j + 2, p)
            pltpu.async_copy(xw[p], acc_sh.at[idxs[p]], ssem[p], add=True)
        return 0
    lax.fori_loop(0, NSUB // 2, step, 0)
    wait_scatter(0)
    wait_scatter(1)

    plsc.subcore_barrier()

    NH = SEGS_TILE // SUB

    def ep_read(h, p):
        seg0 = s * SEGS_TILE + h * SUB
        return pltpu.make_async_copy(acc_sh.at[pl.ds(seg0, SUB)], xw[p], xsem[p])

    def ep_write(h, p):
        seg0 = s * SEGS_TILE + h * SUB
        return pltpu.make_async_copy(xb[p], out_hbm.at[pl.ds(c * SEGS_SC + seg0, SUB)], ssem[p])

    ep_read(0, 0).start()
    ep_read(1, 1).start()
    for h in range(NH):
        p = h & 1
        ep_read(h, p).wait()
        if h >= 2:
            ep_write(h - 2, p).wait()

        @plsc.parallel_loop(0, SUB, unroll=2)
        def nbody(t):
            wrow = xw[p][t, pl.ds(C, L)]
            wv = jnp.full((L,), wrow[0], jnp.float32)
            rv = 1.0 / (wv + 1e-6)
            for f in range(C // L):
                xb[p][t, pl.ds(f * L, L)] = xw[p][t, pl.ds(f * L, L)] * rv

        if h < NH - 2:
            ep_read(h + 2, p).start()
        ep_write(h, p).start()
    ep_write(NH - 2, 0).wait()
    ep_write(NH - 1, 1).wait()


@jax.jit
def _ctm_merge(x2, idx1, tw1):
    mesh = plsc.VectorSubcoreMesh(core_axis_name="c", subcore_axis_name="s")
    run = pl.kernel(
        _sc_kernel,
        out_type=jax.ShapeDtypeStruct((B * CLUSTER, C), jnp.float32),
        mesh=mesh,
        scratch_types=[
            pltpu.VMEM_SHARED((SEGS_SC, W), jnp.float32),
            pltpu.VMEM((SUB, C), jnp.float32),
            pltpu.VMEM((SUB, C), jnp.float32),
            pltpu.VMEM((SUB, W), jnp.float32),
            pltpu.VMEM((SUB, W), jnp.float32),
            pltpu.VMEM((SUB,), jnp.int32),
            pltpu.VMEM((SUB,), jnp.int32),
            pltpu.VMEM((SUB,), jnp.float32),
            pltpu.VMEM((SUB,), jnp.float32),
            pltpu.VMEM((SUB,), jnp.int32),
            pltpu.VMEM((SUB,), jnp.int32),
            pltpu.SemaphoreType.DMA,
            pltpu.SemaphoreType.DMA,
            pltpu.SemaphoreType.DMA,
            pltpu.SemaphoreType.DMA,
        ],
    )
    return run(x2, idx1, tw1)


def kernel(x, idx_cluster, token_weight, cluster_num):
    b, n, c = x.shape
    x2 = x.reshape(b * n, c)
    idx1 = idx_cluster.reshape(b * n)
    tw1 = token_weight.reshape(b * n)
    out = _ctm_merge(x2, idx1, tw1)
    return out.reshape(b, CLUSTER, c)
